# Initial kernel scaffold; baseline (speedup 1.0000x reference)
import jax
import jax.numpy as jnp
from jax.experimental import pallas as pl


def kernel(s, a, w1, b1, w2, b2):
    raise NotImplementedError("write your pallas kernel here")


# speedup vs baseline: 2.8742x; 2.8742x over previous
import jax
import jax.numpy as jnp
from jax import lax
from jax.experimental import pallas as pl
from jax.experimental.pallas import tpu as pltpu


def _make_body(actions: int):
    def _body(s_ref, a_ref, w1s_ref, w1ab_ref, w2_ref, b2_ref, o_ref):
        s = s_ref[...].astype(jnp.bfloat16)
        a = a_ref[...]
        iota = lax.broadcasted_iota(jnp.int32, (a.shape[0], actions), 1)
        onehot = (a == iota).astype(jnp.bfloat16)

        h = jnp.dot(s, w1s_ref[...], preferred_element_type=jnp.float32)
        h = h + jnp.dot(onehot, w1ab_ref[...],
                        preferred_element_type=jnp.float32)
        h = jnp.maximum(h, 0.0).astype(jnp.bfloat16)

        out = jnp.dot(h, w2_ref[...], preferred_element_type=jnp.float32)
        o_ref[...] = out + b2_ref[...]

    return _body


def kernel(s, a, w1, b1, w2, b2):
    T, S = s.shape
    H = w1.shape[1]
    O = w2.shape[1]
    A = w1.shape[0] - S

    b1 = jnp.reshape(b1, (1, H)).astype(jnp.float32)
    b2 = jnp.reshape(b2, (1, O)).astype(jnp.float32)
    w1s = w1[:S].astype(jnp.bfloat16)
    w1ab = (w1[S:] + b1).astype(jnp.bfloat16)
    w2b = w2.astype(jnp.bfloat16)

    TM = 2048
    grid = (pl.cdiv(T, TM),)

    return pl.pallas_call(
        _make_body(A),
        out_shape=jax.ShapeDtypeStruct((T, O), jnp.float32),
        grid=grid,
        in_specs=[
            pl.BlockSpec((TM, S), lambda i: (i, 0)),
            pl.BlockSpec((TM, 1), lambda i: (i, 0)),
            pl.BlockSpec((S, H), lambda i: (0, 0)),
            pl.BlockSpec((A, H), lambda i: (0, 0)),
            pl.BlockSpec((H, O), lambda i: (0, 0)),
            pl.BlockSpec((1, O), lambda i: (0, 0)),
        ],
        out_specs=pl.BlockSpec((TM, O), lambda i: (i, 0)),
        compiler_params=pltpu.CompilerParams(
            dimension_semantics=("parallel",)),
    )(s, a, w1s, w1ab, w2b, b2)


# speedup vs baseline: 3.2584x; 1.1337x over previous
import jax
import jax.numpy as jnp
from jax import lax
from jax.experimental import pallas as pl
from jax.experimental.pallas import tpu as pltpu


def _make_body(actions: int):
    def _body(s_ref, a_ref, w1s_ref, w1ab_ref, w2_ref, b2_ref, o_ref):
        s = s_ref[...].astype(jnp.bfloat16)
        a = a_ref[...]
        iota = lax.broadcasted_iota(jnp.int32, (a.shape[0], actions), 1)
        onehot = (a == iota).astype(jnp.bfloat16)

        h = jnp.dot(s, w1s_ref[...], preferred_element_type=jnp.float32)
        h = h + jnp.dot(onehot, w1ab_ref[...],
                        preferred_element_type=jnp.float32)
        h = jnp.maximum(h, 0.0).astype(jnp.bfloat16)

        out = jnp.dot(h, w2_ref[...], preferred_element_type=jnp.float32)
        o_ref[...] = out + b2_ref[...]

    return _body


def kernel(s, a, w1, b1, w2, b2):
    T, S = s.shape
    H = w1.shape[1]
    O = w2.shape[1]
    A = w1.shape[0] - S

    b1 = jnp.reshape(b1, (1, H)).astype(jnp.float32)
    b2 = jnp.reshape(b2, (1, O)).astype(jnp.float32)
    w1s = w1[:S].astype(jnp.bfloat16)
    w1ab = (w1[S:] + b1).astype(jnp.bfloat16)
    w2b = w2.astype(jnp.bfloat16)

    TM = 4096
    grid = (pl.cdiv(T, TM),)

    return pl.pallas_call(
        _make_body(A),
        out_shape=jax.ShapeDtypeStruct((T, O), jnp.float32),
        grid=grid,
        in_specs=[
            pl.BlockSpec((TM, S), lambda i: (i, 0)),
            pl.BlockSpec((TM, 1), lambda i: (i, 0)),
            pl.BlockSpec((S, H), lambda i: (0, 0)),
            pl.BlockSpec((A, H), lambda i: (0, 0)),
            pl.BlockSpec((H, O), lambda i: (0, 0)),
            pl.BlockSpec((1, O), lambda i: (0, 0)),
        ],
        out_specs=pl.BlockSpec((TM, O), lambda i: (i, 0)),
        compiler_params=pltpu.CompilerParams(
            dimension_semantics=("parallel",)),
    )(s, a, w1s, w1ab, w2b, b2)


# speedup vs baseline: 3.3492x; 1.0279x over previous
import jax
import jax.numpy as jnp
from jax import lax
from jax.experimental import pallas as pl
from jax.experimental.pallas import tpu as pltpu


def _make_body(actions: int):
    def _body(s_ref, a_ref, w1s_ref, w1ab_ref, w2_ref, b2_ref, o_ref):
        s = s_ref[...].astype(jnp.bfloat16)
        a = a_ref[...]
        iota = lax.broadcasted_iota(jnp.int32, (a.shape[0], actions), 1)
        onehot = (a == iota).astype(jnp.bfloat16)

        h = jnp.dot(s, w1s_ref[...], preferred_element_type=jnp.float32)
        h = h + jnp.dot(onehot, w1ab_ref[...],
                        preferred_element_type=jnp.float32)
        h = jnp.maximum(h, 0.0).astype(jnp.bfloat16)

        out = jnp.dot(h, w2_ref[...], preferred_element_type=jnp.float32)
        o_ref[...] = out + b2_ref[...]

    return _body


def kernel(s, a, w1, b1, w2, b2):
    T, S = s.shape
    H = w1.shape[1]
    O = w2.shape[1]
    A = w1.shape[0] - S

    b1 = jnp.reshape(b1, (1, H)).astype(jnp.float32)
    b2 = jnp.reshape(b2, (1, O)).astype(jnp.float32)
    w1s = w1[:S].astype(jnp.bfloat16)
    w1ab = (w1[S:] + b1).astype(jnp.bfloat16)
    w2b = w2.astype(jnp.bfloat16)

    TM = 8192
    grid = (pl.cdiv(T, TM),)

    return pl.pallas_call(
        _make_body(A),
        out_shape=jax.ShapeDtypeStruct((T, O), jnp.float32),
        grid=grid,
        in_specs=[
            pl.BlockSpec((TM, S), lambda i: (i, 0)),
            pl.BlockSpec((TM, 1), lambda i: (i, 0)),
            pl.BlockSpec((S, H), lambda i: (0, 0)),
            pl.BlockSpec((A, H), lambda i: (0, 0)),
            pl.BlockSpec((H, O), lambda i: (0, 0)),
            pl.BlockSpec((1, O), lambda i: (0, 0)),
        ],
        out_specs=pl.BlockSpec((TM, O), lambda i: (i, 0)),
        compiler_params=pltpu.CompilerParams(
            dimension_semantics=("parallel",)),
    )(s, a, w1s, w1ab, w2b, b2)
